# baseline (speedup 1.0000x reference)
import functools
import jax
import jax.numpy as jnp
from jax import lax
from jax.experimental import pallas as pl
from jax.experimental.pallas import tpu as pltpu
from jax.experimental.pallas import tpu_sc as plsc

_PERM4 = (7, 4, 2, 5)

_NC = 2
_NS = 16
_NW = _NC * _NS
_CHUNK = 64
_NBUF = 3


def _make_sc_kernel(n):
    rows_per_w = n // _NW
    n_chunks = rows_per_w // _CHUNK
    mesh = plsc.VectorSubcoreMesh(core_axis_name="c", subcore_axis_name="s")

    @functools.partial(
        pl.kernel,
        mesh=mesh,
        compiler_params=pltpu.CompilerParams(
            use_tc_tiling_on_sc=True,
            disable_bounds_checks=True,
            disable_semaphore_checks=True,
            skip_device_barrier=True,
        ),
        out_type=jax.ShapeDtypeStruct((n, 4, 128), jnp.float32),
        scratch_types=[
            pltpu.VMEM((_NBUF, _CHUNK, 4, 128), jnp.float32),
            pltpu.VMEM_SHARED((_NS, 4, 16, 4, 128), jnp.float32),
            pltpu.SemaphoreType.DMA,
            pltpu.SemaphoreType.DMA,
            pltpu.SemaphoreType.DMA,
            pltpu.SemaphoreType.DMA,
        ],
    )
    def sc_select(x_hbm, out_hbm, buf, spbuf, sem_in, sem_out, sem_spg, sem_sps):
        sid = lax.axis_index("s")
        wid = sid * _NC + lax.axis_index("c")
        base = wid * rows_per_w

        def fire_gathers(ci, b):
            row0 = base + ci * _CHUNK
            for k, p in enumerate(_PERM4):
                pltpu.async_copy(
                    x_hbm.at[pl.ds(row0, _CHUNK), pl.ds(p, 1), :],
                    buf.at[b, :, pl.ds(k, 1), :],
                    sem_in,
                )

        def wait_gathers():
            for _ in range(4):
                pltpu.make_async_copy(
                    x_hbm.at[pl.ds(0, _CHUNK), pl.ds(0, 1), :],
                    buf.at[0, :, pl.ds(0, 1), :],
                    sem_in,
                ).wait()

        def fire_scatter(ci, b):
            row0 = base + ci * _CHUNK
            pltpu.async_copy(buf.at[b], out_hbm.at[pl.ds(row0, _CHUNK)], sem_out)

        def wait_scatter():
            pltpu.make_async_copy(
                buf.at[0], out_hbm.at[pl.ds(0, _CHUNK)], sem_out
            ).wait()

        n_stream = n_chunks - 4
        sp_unit = 16
        sp_units = 16
        sp_base = n_stream * _CHUNK

        def sp_g(u):
            row0 = base + sp_base + u * sp_unit
            for k, p in enumerate(_PERM4):
                pltpu.async_copy(
                    x_hbm.at[pl.ds(row0, sp_unit), pl.ds(p, 1), :],
                    spbuf.at[sid, u % 4, :, pl.ds(k, 1), :],
                    sem_spg,
                )

        def sp_drain_g(ndesc):
            for _ in range(ndesc):
                pltpu.make_async_copy(
                    x_hbm.at[pl.ds(0, sp_unit), pl.ds(0, 1), :],
                    spbuf.at[sid, 0, :, pl.ds(0, 1), :],
                    sem_spg,
                ).wait()

        def sp_s(u):
            row0 = base + sp_base + u * sp_unit
            pltpu.async_copy(
                spbuf.at[sid, u % 4], out_hbm.at[pl.ds(row0, sp_unit)], sem_sps
            )

        def sp_drain_s(ndesc):
            for _ in range(ndesc):
                pltpu.make_async_copy(
                    spbuf.at[sid, 0], out_hbm.at[pl.ds(0, sp_unit)], sem_sps
                ).wait()

        def sp_step(s):
            sp_drain_g(8)
            if s >= 1:
                sp_drain_s(2)
            sp_s(2 * s)
            sp_s(2 * s + 1)
            if s < 7:
                sp_g(2 * s + 2)
                sp_g(2 * s + 3)

        sp_g(0)
        sp_g(1)

        fire_gathers(0, 0)
        fire_gathers(1, 1)
        for ci in range(n_stream):
            b = ci % _NBUF
            if ci + 2 < n_stream:
                if ci >= 1:
                    wait_scatter()
                fire_gathers(ci + 2, (ci + 2) % _NBUF)
            wait_gathers()
            fire_scatter(ci, b)
            sp_step(2 * ci)
            sp_step(2 * ci + 1)
        for _ in range(3):
            wait_scatter()
        sp_drain_s(2)

    return sc_select


def kernel(inputs):
    n = inputs.shape[0]
    out = _make_sc_kernel(n)(inputs)
    return out.reshape(n, 1, 512)


# speedup vs baseline: 1.0351x; 1.0351x over previous
import functools
import jax
import jax.numpy as jnp
from jax import lax
from jax.experimental import pallas as pl
from jax.experimental.pallas import tpu as pltpu
from jax.experimental.pallas import tpu_sc as plsc

_PERM4 = (7, 4, 2, 5)

_NC = 2
_NS = 16
_NW = _NC * _NS
_CHUNK = 64
_NBUF = 3


def _make_sc_kernel(n):
    rows_per_w = n // _NW
    n_chunks = rows_per_w // _CHUNK
    mesh = plsc.VectorSubcoreMesh(core_axis_name="c", subcore_axis_name="s")

    @functools.partial(
        pl.kernel,
        mesh=mesh,
        compiler_params=pltpu.CompilerParams(
            use_tc_tiling_on_sc=True,
            disable_bounds_checks=True,
            disable_semaphore_checks=True,
            skip_device_barrier=True,
        ),
        out_type=jax.ShapeDtypeStruct((n, 4, 128), jnp.float32),
        scratch_types=[
            pltpu.VMEM((_NBUF, _CHUNK, 4, 128), jnp.float32),
            pltpu.VMEM_SHARED((_NS, 4, 16, 4, 128), jnp.float32),
            pltpu.SemaphoreType.DMA,
            pltpu.SemaphoreType.DMA,
            pltpu.SemaphoreType.DMA,
            pltpu.SemaphoreType.DMA,
        ],
    )
    def sc_select(x_hbm, out_hbm, buf, spbuf, sem_in, sem_out, sem_spg, sem_sps):
        sid = lax.axis_index("s")
        wid = sid * _NC + lax.axis_index("c")
        base = wid * rows_per_w

        def fire_gathers(ci, b):
            row0 = base + ci * _CHUNK
            for k, p in enumerate(_PERM4):
                pltpu.async_copy(
                    x_hbm.at[pl.ds(row0, _CHUNK), pl.ds(p, 1), :],
                    buf.at[b, :, pl.ds(k, 1), :],
                    sem_in,
                )

        def wait_gathers():
            for _ in range(4):
                pltpu.make_async_copy(
                    x_hbm.at[pl.ds(0, _CHUNK), pl.ds(0, 1), :],
                    buf.at[0, :, pl.ds(0, 1), :],
                    sem_in,
                ).wait()

        def fire_scatter(ci, b):
            row0 = base + ci * _CHUNK
            pltpu.async_copy(buf.at[b], out_hbm.at[pl.ds(row0, _CHUNK)], sem_out)

        def wait_scatter():
            pltpu.make_async_copy(
                buf.at[0], out_hbm.at[pl.ds(0, _CHUNK)], sem_out
            ).wait()

        n_stream = n_chunks - 3
        sp_unit = 16
        sp_units = 12
        sp_base = n_stream * _CHUNK

        def sp_g(u):
            row0 = base + sp_base + u * sp_unit
            for k, p in enumerate(_PERM4):
                pltpu.async_copy(
                    x_hbm.at[pl.ds(row0, sp_unit), pl.ds(p, 1), :],
                    spbuf.at[sid, u % 4, :, pl.ds(k, 1), :],
                    sem_spg,
                )

        def sp_drain_g(ndesc):
            for _ in range(ndesc):
                pltpu.make_async_copy(
                    x_hbm.at[pl.ds(0, sp_unit), pl.ds(0, 1), :],
                    spbuf.at[sid, 0, :, pl.ds(0, 1), :],
                    sem_spg,
                ).wait()

        def sp_s(u):
            row0 = base + sp_base + u * sp_unit
            pltpu.async_copy(
                spbuf.at[sid, u % 4], out_hbm.at[pl.ds(row0, sp_unit)], sem_sps
            )

        def sp_drain_s(ndesc):
            for _ in range(ndesc):
                pltpu.make_async_copy(
                    spbuf.at[sid, 0], out_hbm.at[pl.ds(0, sp_unit)], sem_sps
                ).wait()

        def sp_step(s):
            sp_drain_g(8)
            if s >= 1:
                sp_drain_s(2)
            sp_s(2 * s)
            sp_s(2 * s + 1)
            if s < sp_units // 2 - 1:
                sp_g(2 * s + 2)
                sp_g(2 * s + 3)

        sp_g(0)
        sp_g(1)

        fire_gathers(0, 0)
        fire_gathers(1, 1)
        for ci in range(n_stream):
            b = ci % _NBUF
            if ci + 2 < n_stream:
                if ci >= 1:
                    wait_scatter()
                fire_gathers(ci + 2, (ci + 2) % _NBUF)
            wait_gathers()
            fire_scatter(ci, b)
            if ci == 0:
                sp_step(0)
                sp_step(1)
            else:
                sp_step(ci + 1)
        for _ in range(3):
            wait_scatter()
        sp_drain_s(2)

    return sc_select


def kernel(inputs):
    n = inputs.shape[0]
    out = _make_sc_kernel(n)(inputs)
    return out.reshape(n, 1, 512)


# speedup vs baseline: 1.0570x; 1.0212x over previous
import functools
import jax
import jax.numpy as jnp
from jax import lax
from jax.experimental import pallas as pl
from jax.experimental.pallas import tpu as pltpu
from jax.experimental.pallas import tpu_sc as plsc

_PERM4 = (7, 4, 2, 5)

_NC = 2
_NS = 16
_NW = _NC * _NS
_CHUNK = 64
_NBUF = 3


def _make_sc_kernel(n):
    rows_per_w = n // _NW
    n_chunks = rows_per_w // _CHUNK
    mesh = plsc.VectorSubcoreMesh(core_axis_name="c", subcore_axis_name="s")

    @functools.partial(
        pl.kernel,
        mesh=mesh,
        compiler_params=pltpu.CompilerParams(
            use_tc_tiling_on_sc=True,
            disable_bounds_checks=True,
            disable_semaphore_checks=True,
            skip_device_barrier=True,
        ),
        out_type=jax.ShapeDtypeStruct((n, 4, 128), jnp.float32),
        scratch_types=[
            pltpu.VMEM((_NBUF, _CHUNK, 4, 128), jnp.float32),
            pltpu.VMEM_SHARED((_NS, 2, _CHUNK // 2, 4, 128), jnp.float32),
            pltpu.SemaphoreType.DMA,
            pltpu.SemaphoreType.DMA,
            pltpu.SemaphoreType.DMA,
            pltpu.SemaphoreType.DMA,
        ],
    )
    def sc_select(x_hbm, out_hbm, buf, spbuf, sem_in, sem_out, sem_spg, sem_sps):
        sid = lax.axis_index("s")
        wid = sid * _NC + lax.axis_index("c")
        base = wid * rows_per_w

        def fire_gathers(ci, b):
            row0 = base + ci * _CHUNK
            for k, p in enumerate(_PERM4):
                pltpu.async_copy(
                    x_hbm.at[pl.ds(row0, _CHUNK), pl.ds(p, 1), :],
                    buf.at[b, :, pl.ds(k, 1), :],
                    sem_in,
                )

        def wait_gathers():
            for _ in range(4):
                pltpu.make_async_copy(
                    x_hbm.at[pl.ds(0, _CHUNK), pl.ds(0, 1), :],
                    buf.at[0, :, pl.ds(0, 1), :],
                    sem_in,
                ).wait()

        def fire_scatter(ci, b):
            row0 = base + ci * _CHUNK
            pltpu.async_copy(buf.at[b], out_hbm.at[pl.ds(row0, _CHUNK)], sem_out)

        def wait_scatter():
            pltpu.make_async_copy(
                buf.at[0], out_hbm.at[pl.ds(0, _CHUNK)], sem_out
            ).wait()

        n_stream = n_chunks - 2
        sp_half = _CHUNK // 2
        sp_base = n_stream * _CHUNK

        def sp_fire_gathers(j):
            row0 = base + sp_base + j * sp_half
            for k, p in enumerate(_PERM4):
                pltpu.async_copy(
                    x_hbm.at[pl.ds(row0, sp_half), pl.ds(p, 1), :],
                    spbuf.at[sid, j % 2, :, pl.ds(k, 1), :],
                    sem_spg,
                )

        def sp_wait_gathers():
            for _ in range(4):
                pltpu.make_async_copy(
                    x_hbm.at[pl.ds(0, sp_half), pl.ds(0, 1), :],
                    spbuf.at[sid, 0, :, pl.ds(0, 1), :],
                    sem_spg,
                ).wait()

        def sp_fire_scatter(j):
            row0 = base + sp_base + j * sp_half
            pltpu.async_copy(
                spbuf.at[sid, j % 2], out_hbm.at[pl.ds(row0, sp_half)], sem_sps
            )

        def sp_wait_scatter():
            pltpu.make_async_copy(
                spbuf.at[sid, 0], out_hbm.at[pl.ds(0, sp_half)], sem_sps
            ).wait()

        sp_fire_gathers(0)
        sp_fire_gathers(1)

        fire_gathers(0, 0)
        fire_gathers(1, 1)
        for ci in range(n_stream):
            b = ci % _NBUF
            if ci + 2 < n_stream:
                if ci >= 1:
                    wait_scatter()
                fire_gathers(ci + 2, (ci + 2) % _NBUF)
            wait_gathers()
            fire_scatter(ci, b)
            if ci == 1:
                sp_wait_gathers()
                sp_wait_gathers()
                sp_fire_scatter(0)
                sp_fire_scatter(1)
            elif ci == 3:
                sp_wait_scatter()
                sp_wait_scatter()
                sp_fire_gathers(2)
                sp_fire_gathers(3)
            elif ci == 5:
                sp_wait_gathers()
                sp_wait_gathers()
                sp_fire_scatter(2)
                sp_fire_scatter(3)
        for _ in range(3):
            wait_scatter()
        sp_wait_scatter()
        sp_wait_scatter()

    return sc_select


def kernel(inputs):
    n = inputs.shape[0]
    out = _make_sc_kernel(n)(inputs)
    return out.reshape(n, 1, 512)
